# Initial kernel scaffold; baseline (speedup 1.0000x reference)
import jax
import jax.numpy as jnp
from jax.experimental import pallas as pl


def kernel(node_tensor, edge_tensor, Wn, bn, We, be, msg_W1, msg_b1, msg_W2, msg_b2, upd_W1, upd_b1, upd_W2, upd_b2, ln_g, ln_b, out_W1, out_b1, out_W2, out_b2, gr_W1, gr_b1, gr_W2, gr_b2, edge_index):
    raise NotImplementedError("write your pallas kernel here")


# speedup vs baseline: 1.8194x; 1.8194x over previous
import functools

import jax
import jax.numpy as jnp
from jax import lax
from jax.experimental import pallas as pl
from jax.experimental.pallas import tpu as pltpu
from jax.experimental.pallas import tpu_sc as plsc

N = 10000
H = 128
NI = 7
EI = 4
L = 4
OUT = 8

NC = 2
NS = 16
NW = NC * NS
CH = 128
NP = 10240
RPT = NP // NS


def _sc_gather(hs_t, hd_t, src2, dst2, e_pad):
    K = e_pad // (NW * CH)
    mesh = plsc.VectorSubcoreMesh(
        core_axis_name="c", subcore_axis_name="s", num_cores=NC, num_subcores=NS)

    @functools.partial(
        pl.kernel, mesh=mesh,
        out_type=[jax.ShapeDtypeStruct((e_pad, H), jnp.float32),
                  jax.ShapeDtypeStruct((e_pad, H), jnp.float32)],
        scratch_types=[pltpu.VMEM((K, CH), jnp.int32),
                       pltpu.VMEM((K, CH), jnp.int32),
                       pltpu.VMEM((CH, H), jnp.float32),
                       pltpu.VMEM((CH, H), jnp.float32),
                       pltpu.SemaphoreType.DMA,
                       pltpu.SemaphoreType.DMA])
    def k(hs_hbm, hd_hbm, src_hbm, dst_hbm, gs_hbm, gd_hbm,
          idx_s, idx_d, rows_s, rows_d, sem_s, sem_d):
        wid = lax.axis_index("s") * NC + lax.axis_index("c")
        base = wid * (K * CH)
        pltpu.sync_copy(src_hbm.at[wid], idx_s)
        pltpu.sync_copy(dst_hbm.at[wid], idx_d)

        def chunk(c, carry):
            cp_s = pltpu.async_copy(hs_hbm.at[idx_s.at[c]], rows_s, sem_s)
            cp_d = pltpu.async_copy(hd_hbm.at[idx_d.at[c]], rows_d, sem_d)
            cp_s.wait()
            cp_d.wait()
            pltpu.sync_copy(rows_s, gs_hbm.at[pl.ds(base + c * CH, CH)])
            pltpu.sync_copy(rows_d, gd_hbm.at[pl.ds(base + c * CH, CH)])
            return carry

        lax.fori_loop(0, K, chunk, 0)

    return k(hs_t, hd_t, src2, dst2)


def _sc_scatter(m, dst2s, zero_rows, e_pad):
    K = e_pad // (NW * CH)
    mesh = plsc.VectorSubcoreMesh(
        core_axis_name="c", subcore_axis_name="s", num_cores=NC, num_subcores=NS)

    @functools.partial(
        pl.kernel, mesh=mesh,
        out_type=[jax.ShapeDtypeStruct((NC, NP, H), jnp.float32)],
        scratch_types=[pltpu.VMEM((K, CH), jnp.int32),
                       pltpu.VMEM((CH, H), jnp.float32),
                       pltpu.VMEM_SHARED((NP, H), jnp.float32)])
    def k(m_hbm, dst_hbm, zero_hbm, agg_hbm, idx_d, mbuf, acc_sp):
        cid = lax.axis_index("c")
        sid = lax.axis_index("s")
        wid = sid * NC + cid
        base = wid * (K * CH)
        tile0 = sid * RPT
        pltpu.sync_copy(dst_hbm.at[wid], idx_d)
        for j in range(RPT // CH):
            pltpu.sync_copy(zero_hbm, acc_sp.at[pl.ds(tile0 + j * CH, CH)])
        plsc.subcore_barrier()

        def chunk(c, carry):
            pltpu.sync_copy(m_hbm.at[pl.ds(base + c * CH, CH)], mbuf)
            pltpu.sync_copy(mbuf, acc_sp.at[idx_d.at[c]], add=True)
            return carry

        lax.fori_loop(0, K, chunk, 0)
        plsc.subcore_barrier()
        for j in range(RPT // CH):
            sl = pl.ds(tile0 + j * CH, CH)
            pltpu.sync_copy(acc_sp.at[sl], agg_hbm.at[cid, sl])

    return k(m, dst2s, zero_rows)


def _sc_counts(dst2s, zero_rows, ones_rows):
    K = dst2s.shape[1]
    mesh = plsc.VectorSubcoreMesh(
        core_axis_name="c", subcore_axis_name="s", num_cores=NC, num_subcores=NS)

    @functools.partial(
        pl.kernel, mesh=mesh,
        out_type=[jax.ShapeDtypeStruct((NC, NP, H), jnp.float32)],
        scratch_types=[pltpu.VMEM((K, CH), jnp.int32),
                       pltpu.VMEM((CH, H), jnp.float32),
                       pltpu.VMEM_SHARED((NP, H), jnp.float32)])
    def k(dst_hbm, zero_hbm, ones_hbm, cnt_hbm, idx_d, obuf, acc_sp):
        cid = lax.axis_index("c")
        sid = lax.axis_index("s")
        wid = sid * NC + cid
        tile0 = sid * RPT
        pltpu.sync_copy(dst_hbm.at[wid], idx_d)
        pltpu.sync_copy(ones_hbm, obuf)
        for j in range(RPT // CH):
            pltpu.sync_copy(zero_hbm, acc_sp.at[pl.ds(tile0 + j * CH, CH)])
        plsc.subcore_barrier()

        def chunk(c, carry):
            pltpu.sync_copy(obuf, acc_sp.at[idx_d.at[c]], add=True)
            return carry

        lax.fori_loop(0, K, chunk, 0)
        plsc.subcore_barrier()
        for j in range(RPT // CH):
            sl = pl.ds(tile0 + j * CH, CH)
            pltpu.sync_copy(acc_sp.at[sl], cnt_hbm.at[cid, sl])

    return k(dst2s, zero_rows, ones_rows)


def _tc_encode_nodes(nt8, Wn8, bn):
    def body(x_ref, w_ref, b_ref, o_ref):
        o_ref[...] = jnp.dot(x_ref[...], w_ref[...],
                             preferred_element_type=jnp.float32) + b_ref[...]
    return pl.pallas_call(
        body, out_shape=jax.ShapeDtypeStruct((N, H), jnp.float32))(nt8, Wn8, bn[None, :])


def _tc_encode_edges(et8, We8, be, e_pad):
    BE = 2048
    def body(x_ref, w_ref, b_ref, o_ref):
        o_ref[...] = jnp.dot(x_ref[...], w_ref[...],
                             preferred_element_type=jnp.float32) + b_ref[...]
    return pl.pallas_call(
        body,
        grid=(e_pad // BE,),
        in_specs=[pl.BlockSpec((BE, 8), lambda i: (i, 0)),
                  pl.BlockSpec((8, H), lambda i: (0, 0)),
                  pl.BlockSpec((1, H), lambda i: (0, 0))],
        out_specs=pl.BlockSpec((BE, H), lambda i: (i, 0)),
        out_shape=jax.ShapeDtypeStruct((e_pad, H), jnp.float32))(et8, We8, be[None, :])


def _tc_proj(h, Wa, Wb):
    def body(h_ref, wa_ref, wb_ref, hs_ref, hd_ref):
        hv = h_ref[...]
        hs_ref[...] = jnp.dot(hv, wa_ref[...], preferred_element_type=jnp.float32)
        hd_ref[...] = jnp.dot(hv, wb_ref[...], preferred_element_type=jnp.float32)
    return pl.pallas_call(
        body,
        out_shape=[jax.ShapeDtypeStruct((N, H), jnp.float32),
                   jax.ShapeDtypeStruct((N, H), jnp.float32)])(h, Wa, Wb)


def _tc_edge_mlp(gs, gd, ef1, W1c, W2, b1, b2, e_pad):
    BE = 2048
    def body(gs_ref, gd_ref, ef_ref, w1_ref, w2_ref, b1_ref, b2_ref, m_ref):
        pre = (gs_ref[...] + gd_ref[...]
               + jnp.dot(ef_ref[...], w1_ref[...],
                         preferred_element_type=jnp.float32) + b1_ref[...])
        m_ref[...] = jnp.dot(jnp.maximum(pre, 0.0), w2_ref[...],
                             preferred_element_type=jnp.float32) + b2_ref[...]
    return pl.pallas_call(
        body,
        grid=(e_pad // BE,),
        in_specs=[pl.BlockSpec((BE, H), lambda i: (i, 0)),
                  pl.BlockSpec((BE, H), lambda i: (i, 0)),
                  pl.BlockSpec((BE, H), lambda i: (i, 0)),
                  pl.BlockSpec((H, H), lambda i: (0, 0)),
                  pl.BlockSpec((H, H), lambda i: (0, 0)),
                  pl.BlockSpec((1, H), lambda i: (0, 0)),
                  pl.BlockSpec((1, H), lambda i: (0, 0))],
        out_specs=pl.BlockSpec((BE, H), lambda i: (i, 0)),
        out_shape=jax.ShapeDtypeStruct((e_pad, H), jnp.float32))(
            gs, gd, ef1, W1c, W2, b1[None, :], b2[None, :])


def _tc_update(h, parts, cnt_parts, W1h, W1a, b1, W2, b2, g, b):
    def body(h_ref, p_ref, c_ref, w1h_ref, w1a_ref, b1_ref, w2_ref, b2_ref,
             g_ref, bb_ref, o_ref):
        hv = h_ref[...]
        agg = p_ref[0, :N, :] + p_ref[1, :N, :]
        cnt = c_ref[0, :N, 0:1] + c_ref[1, :N, 0:1]
        agg = agg * (1.0 / jnp.maximum(cnt, 1.0))
        pre = (jnp.dot(hv, w1h_ref[...], preferred_element_type=jnp.float32)
               + jnp.dot(agg, w1a_ref[...], preferred_element_type=jnp.float32)
               + b1_ref[...])
        u = jnp.dot(jnp.maximum(pre, 0.0), w2_ref[...],
                    preferred_element_type=jnp.float32) + b2_ref[...]
        x = hv + u
        mu = jnp.mean(x, axis=-1, keepdims=True)
        var = jnp.mean((x - mu) ** 2, axis=-1, keepdims=True)
        o_ref[...] = (x - mu) * lax.rsqrt(var + 1e-5) * g_ref[...] + bb_ref[...]
    return pl.pallas_call(
        body,
        out_shape=jax.ShapeDtypeStruct((N, H), jnp.float32))(
            h, parts, cnt_parts, W1h, W1a, b1[None, :], W2, b2[None, :],
            g[None, :], b[None, :])


def _tc_heads(h, oW1, ob1, oW2p, ob2p, gW1, gb1, gW2p, gb2p):
    def body(h_ref, ow1_ref, ob1_ref, ow2_ref, ob2_ref,
             gw1_ref, gb1_ref, gw2_ref, gb2_ref, no_ref, go_ref):
        hv = h_ref[...]
        t = jnp.maximum(jnp.dot(hv, ow1_ref[...],
                                preferred_element_type=jnp.float32)
                        + ob1_ref[...], 0.0)
        no_ref[...] = jnp.dot(t, ow2_ref[...],
                              preferred_element_type=jnp.float32) + ob2_ref[...]
        gm = jnp.sum(hv, axis=0, keepdims=True) * (1.0 / N)
        tg = jnp.maximum(jnp.dot(gm, gw1_ref[...],
                                 preferred_element_type=jnp.float32)
                         + gb1_ref[...], 0.0)
        go_ref[...] = jnp.dot(tg, gw2_ref[...],
                              preferred_element_type=jnp.float32) + gb2_ref[...]
    return pl.pallas_call(
        body,
        out_shape=[jax.ShapeDtypeStruct((N, H), jnp.float32),
                   jax.ShapeDtypeStruct((1, H), jnp.float32)])(
            h, oW1, ob1[None, :], oW2p, ob2p[None, :],
            gW1, gb1[None, :], gW2p, gb2p[None, :])


def kernel(node_tensor, edge_tensor, Wn, bn, We, be, msg_W1, msg_b1, msg_W2,
           msg_b2, upd_W1, upd_b1, upd_W2, upd_b2, ln_g, ln_b, out_W1, out_b1,
           out_W2, out_b2, gr_W1, gr_b1, gr_W2, gr_b2, edge_index):
    E = edge_tensor.shape[0]
    e_pad = ((E + NW * CH - 1) // (NW * CH)) * (NW * CH)

    src = edge_index[0].astype(jnp.int32)
    dst = edge_index[1].astype(jnp.int32)
    src2 = jnp.pad(src, (0, e_pad - E)).reshape(NW, -1, CH)
    dst2 = jnp.pad(dst, (0, e_pad - E)).reshape(NW, -1, CH)
    dst2s = jnp.pad(dst, (0, e_pad - E), constant_values=N).reshape(NW, -1, CH)

    nt8 = jnp.pad(node_tensor, ((0, 0), (0, 8 - NI)))
    Wn8 = jnp.pad(Wn, ((0, 8 - NI), (0, 0)))
    et8 = jnp.pad(edge_tensor, ((0, e_pad - E), (0, 8 - EI)))
    We8 = jnp.pad(We, ((0, 8 - EI), (0, 0)))

    zero_rows = jnp.zeros((CH, H), jnp.float32)
    ones_rows = jnp.ones((CH, H), jnp.float32)

    oW2p = jnp.pad(out_W2, ((0, 0), (0, H - OUT)))
    ob2p = jnp.pad(out_b2, (0, H - OUT))
    gW2p = jnp.pad(gr_W2, ((0, 0), (0, H - OUT)))
    gb2p = jnp.pad(gr_b2, (0, H - OUT))

    h = _tc_encode_nodes(nt8, Wn8, bn)
    ef1 = _tc_encode_edges(et8, We8, be, e_pad)
    (cnt_parts,) = _sc_counts(dst2s, zero_rows, ones_rows)
    cnt_parts = cnt_parts[:, :, :8]

    for l in range(L):
        W1a = msg_W1[l, :H, :]
        W1b = msg_W1[l, H:2 * H, :]
        W1c = msg_W1[l, 2 * H:, :]
        hs_t, hd_t = _tc_proj(h, W1a, W1b)
        gs, gd = _sc_gather(hs_t, hd_t, src2, dst2, e_pad)
        m = _tc_edge_mlp(gs, gd, ef1, W1c, msg_W2[l], msg_b1[l], msg_b2[l],
                         e_pad)
        (parts,) = _sc_scatter(m, dst2s, zero_rows, e_pad)
        h = _tc_update(h, parts, cnt_parts, upd_W1[l, :H, :],
                       upd_W1[l, H:, :], upd_b1[l], upd_W2[l], upd_b2[l],
                       ln_g[l], ln_b[l])

    no_pad, go_pad = _tc_heads(h, out_W1, out_b1, oW2p, ob2p,
                               gr_W1, gr_b1, gW2p, gb2p)
    return no_pad[:, :OUT], go_pad[0, :OUT]


# speedup vs baseline: 2.1168x; 1.1635x over previous
import functools

import jax
import jax.numpy as jnp
from jax import lax
from jax.experimental import pallas as pl
from jax.experimental.pallas import tpu as pltpu
from jax.experimental.pallas import tpu_sc as plsc

N = 10000
H = 128
NI = 7
EI = 4
L = 4
OUT = 8

NC = 2
NS = 16
NW = NC * NS
CH = 128
NP = 10240
RPT = NP // NS


def _sc_gather(hs_t, hd_t, src2, dst2, e_pad):
    K = e_pad // (NW * CH)
    mesh = plsc.VectorSubcoreMesh(
        core_axis_name="c", subcore_axis_name="s", num_cores=NC, num_subcores=NS)

    @functools.partial(
        pl.kernel, mesh=mesh,
        out_type=[jax.ShapeDtypeStruct((e_pad, H), jnp.float32),
                  jax.ShapeDtypeStruct((e_pad, H), jnp.float32)],
        scratch_types=[pltpu.VMEM((K, CH), jnp.int32),
                       pltpu.VMEM((K, CH), jnp.int32),
                       pltpu.VMEM((2, CH, H), jnp.float32),
                       pltpu.VMEM((2, CH, H), jnp.float32),
                       [pltpu.SemaphoreType.DMA] * 2,
                       [pltpu.SemaphoreType.DMA] * 2,
                       [pltpu.SemaphoreType.DMA] * 2,
                       [pltpu.SemaphoreType.DMA] * 2])
    def k(hs_hbm, hd_hbm, src_hbm, dst_hbm, gs_hbm, gd_hbm,
          idx_s, idx_d, rows_s, rows_d, gsem_s, gsem_d, wsem_s, wsem_d):
        wid = lax.axis_index("s") * NC + lax.axis_index("c")
        base = wid * (K * CH)
        pltpu.sync_copy(src_hbm.at[wid], idx_s)
        pltpu.sync_copy(dst_hbm.at[wid], idx_d)

        def start_g(c, b):
            pltpu.async_copy(hs_hbm.at[idx_s.at[c]], rows_s.at[b], gsem_s[b])
            pltpu.async_copy(hd_hbm.at[idx_d.at[c]], rows_d.at[b], gsem_d[b])

        def wait_g(b):
            pltpu.make_async_copy(hs_hbm.at[idx_s.at[0]], rows_s.at[b],
                                  gsem_s[b]).wait()
            pltpu.make_async_copy(hd_hbm.at[idx_d.at[0]], rows_d.at[b],
                                  gsem_d[b]).wait()

        def start_w(c, b):
            sl = pl.ds(base + c * CH, CH)
            pltpu.async_copy(rows_s.at[b], gs_hbm.at[sl], wsem_s[b])
            pltpu.async_copy(rows_d.at[b], gd_hbm.at[sl], wsem_d[b])

        def wait_w(b):
            sl = pl.ds(base, CH)
            pltpu.make_async_copy(rows_s.at[b], gs_hbm.at[sl], wsem_s[b]).wait()
            pltpu.make_async_copy(rows_d.at[b], gd_hbm.at[sl], wsem_d[b]).wait()

        start_g(0, 0)

        def body(gg, carry):
            for b in (0, 1):
                c = gg * 2 + b
                nb = 1 - b

                @pl.when(c >= 1)
                def _():
                    wait_w(nb)

                @pl.when(c + 1 < K)
                def _():
                    start_g(c + 1, nb)

                wait_g(b)
                start_w(c, b)
            return carry

        lax.fori_loop(0, K // 2, body, 0)
        wait_w(1)

    return k(hs_t, hd_t, src2, dst2)


def _sc_scatter(m, dst2s, zero_rows, e_pad):
    K = e_pad // (NW * CH)
    mesh = plsc.VectorSubcoreMesh(
        core_axis_name="c", subcore_axis_name="s", num_cores=NC, num_subcores=NS)

    @functools.partial(
        pl.kernel, mesh=mesh,
        out_type=[jax.ShapeDtypeStruct((NC, NP, H), jnp.float32)],
        scratch_types=[pltpu.VMEM((K, CH), jnp.int32),
                       pltpu.VMEM((2, CH, H), jnp.float32),
                       pltpu.VMEM_SHARED((NP, H), jnp.float32),
                       [pltpu.SemaphoreType.DMA] * 2,
                       [pltpu.SemaphoreType.DMA] * 2])
    def k(m_hbm, dst_hbm, zero_hbm, agg_hbm, idx_d, mbuf, acc_sp, rsem, asem):
        cid = lax.axis_index("c")
        sid = lax.axis_index("s")
        wid = sid * NC + cid
        base = wid * (K * CH)
        tile0 = sid * RPT
        pltpu.sync_copy(dst_hbm.at[wid], idx_d)
        for j in range(RPT // CH):
            pltpu.sync_copy(zero_hbm, acc_sp.at[pl.ds(tile0 + j * CH, CH)])
        plsc.subcore_barrier()

        def start_r(c, b):
            pltpu.async_copy(m_hbm.at[pl.ds(base + c * CH, CH)], mbuf.at[b],
                             rsem[b])

        def wait_r(b):
            pltpu.make_async_copy(m_hbm.at[pl.ds(base, CH)], mbuf.at[b],
                                  rsem[b]).wait()

        def start_a(c, b):
            pltpu.async_copy(mbuf.at[b], acc_sp.at[idx_d.at[c]], asem[b],
                             add=True)

        def wait_a(b):
            pltpu.make_async_copy(mbuf.at[b], acc_sp.at[idx_d.at[0]],
                                  asem[b]).wait()

        start_r(0, 0)

        def body(gg, carry):
            for b in (0, 1):
                c = gg * 2 + b
                nb = 1 - b

                @pl.when(c >= 1)
                def _():
                    wait_a(nb)

                @pl.when(c + 1 < K)
                def _():
                    start_r(c + 1, nb)

                wait_r(b)
                start_a(c, b)
            return carry

        lax.fori_loop(0, K // 2, body, 0)
        wait_a(1)
        plsc.subcore_barrier()
        for j in range(RPT // CH):
            sl = pl.ds(tile0 + j * CH, CH)
            pltpu.sync_copy(acc_sp.at[sl], agg_hbm.at[cid, sl])

    return k(m, dst2s, zero_rows)


def _sc_counts(dst2s, zero_rows, ones_rows):
    K = dst2s.shape[1]
    mesh = plsc.VectorSubcoreMesh(
        core_axis_name="c", subcore_axis_name="s", num_cores=NC, num_subcores=NS)

    @functools.partial(
        pl.kernel, mesh=mesh,
        out_type=[jax.ShapeDtypeStruct((NC, NP, H), jnp.float32)],
        scratch_types=[pltpu.VMEM((K, CH), jnp.int32),
                       pltpu.VMEM((CH, H), jnp.float32),
                       pltpu.VMEM_SHARED((NP, H), jnp.float32),
                       pltpu.SemaphoreType.DMA])
    def k(dst_hbm, zero_hbm, ones_hbm, cnt_hbm, idx_d, obuf, acc_sp, asem):
        cid = lax.axis_index("c")
        sid = lax.axis_index("s")
        wid = sid * NC + cid
        tile0 = sid * RPT
        pltpu.sync_copy(dst_hbm.at[wid], idx_d)
        pltpu.sync_copy(ones_hbm, obuf)
        for j in range(RPT // CH):
            pltpu.sync_copy(zero_hbm, acc_sp.at[pl.ds(tile0 + j * CH, CH)])
        plsc.subcore_barrier()

        def chunk(c, carry):
            pltpu.async_copy(obuf, acc_sp.at[idx_d.at[c]], asem, add=True)
            return carry

        lax.fori_loop(0, K, chunk, 0)

        def drain(c, carry):
            pltpu.make_async_copy(obuf, acc_sp.at[idx_d.at[0]],
                                  asem).wait()
            return carry

        lax.fori_loop(0, K, drain, 0)
        plsc.subcore_barrier()
        for j in range(RPT // CH):
            sl = pl.ds(tile0 + j * CH, CH)
            pltpu.sync_copy(acc_sp.at[sl], cnt_hbm.at[cid, sl])

    return k(dst2s, zero_rows, ones_rows)


def _tc_encode_nodes(nt8, Wn8, bn):
    def body(x_ref, w_ref, b_ref, o_ref):
        o_ref[...] = jnp.dot(x_ref[...], w_ref[...],
                             preferred_element_type=jnp.float32) + b_ref[...]
    return pl.pallas_call(
        body, out_shape=jax.ShapeDtypeStruct((N, H), jnp.float32))(nt8, Wn8, bn[None, :])


def _tc_encode_edges(et8, We8, be, e_pad):
    BE = 2048
    def body(x_ref, w_ref, b_ref, o_ref):
        o_ref[...] = jnp.dot(x_ref[...], w_ref[...],
                             preferred_element_type=jnp.float32) + b_ref[...]
    return pl.pallas_call(
        body,
        grid=(e_pad // BE,),
        in_specs=[pl.BlockSpec((BE, 8), lambda i: (i, 0)),
                  pl.BlockSpec((8, H), lambda i: (0, 0)),
                  pl.BlockSpec((1, H), lambda i: (0, 0))],
        out_specs=pl.BlockSpec((BE, H), lambda i: (i, 0)),
        out_shape=jax.ShapeDtypeStruct((e_pad, H), jnp.float32))(et8, We8, be[None, :])


def _tc_proj(h, Wa, Wb):
    def body(h_ref, wa_ref, wb_ref, hs_ref, hd_ref):
        hv = h_ref[...]
        hs_ref[...] = jnp.dot(hv, wa_ref[...], preferred_element_type=jnp.float32)
        hd_ref[...] = jnp.dot(hv, wb_ref[...], preferred_element_type=jnp.float32)
    return pl.pallas_call(
        body,
        out_shape=[jax.ShapeDtypeStruct((N, H), jnp.float32),
                   jax.ShapeDtypeStruct((N, H), jnp.float32)])(h, Wa, Wb)


def _tc_edge_mlp(gs, gd, ef1, W1c, W2, b1, b2, e_pad):
    BE = 2048
    def body(gs_ref, gd_ref, ef_ref, w1_ref, w2_ref, b1_ref, b2_ref, m_ref):
        pre = (gs_ref[...] + gd_ref[...]
               + jnp.dot(ef_ref[...], w1_ref[...],
                         preferred_element_type=jnp.float32) + b1_ref[...])
        m_ref[...] = jnp.dot(jnp.maximum(pre, 0.0), w2_ref[...],
                             preferred_element_type=jnp.float32) + b2_ref[...]
    return pl.pallas_call(
        body,
        grid=(e_pad // BE,),
        in_specs=[pl.BlockSpec((BE, H), lambda i: (i, 0)),
                  pl.BlockSpec((BE, H), lambda i: (i, 0)),
                  pl.BlockSpec((BE, H), lambda i: (i, 0)),
                  pl.BlockSpec((H, H), lambda i: (0, 0)),
                  pl.BlockSpec((H, H), lambda i: (0, 0)),
                  pl.BlockSpec((1, H), lambda i: (0, 0)),
                  pl.BlockSpec((1, H), lambda i: (0, 0))],
        out_specs=pl.BlockSpec((BE, H), lambda i: (i, 0)),
        out_shape=jax.ShapeDtypeStruct((e_pad, H), jnp.float32))(
            gs, gd, ef1, W1c, W2, b1[None, :], b2[None, :])


def _tc_update(h, parts, cnt_parts, W1h, W1a, b1, W2, b2, g, b):
    def body(h_ref, p_ref, c_ref, w1h_ref, w1a_ref, b1_ref, w2_ref, b2_ref,
             g_ref, bb_ref, o_ref):
        hv = h_ref[...]
        agg = p_ref[0, :N, :] + p_ref[1, :N, :]
        cnt = c_ref[0, :N, 0:1] + c_ref[1, :N, 0:1]
        agg = agg * (1.0 / jnp.maximum(cnt, 1.0))
        pre = (jnp.dot(hv, w1h_ref[...], preferred_element_type=jnp.float32)
               + jnp.dot(agg, w1a_ref[...], preferred_element_type=jnp.float32)
               + b1_ref[...])
        u = jnp.dot(jnp.maximum(pre, 0.0), w2_ref[...],
                    preferred_element_type=jnp.float32) + b2_ref[...]
        x = hv + u
        mu = jnp.mean(x, axis=-1, keepdims=True)
        var = jnp.mean((x - mu) ** 2, axis=-1, keepdims=True)
        o_ref[...] = (x - mu) * lax.rsqrt(var + 1e-5) * g_ref[...] + bb_ref[...]
    return pl.pallas_call(
        body,
        out_shape=jax.ShapeDtypeStruct((N, H), jnp.float32))(
            h, parts, cnt_parts, W1h, W1a, b1[None, :], W2, b2[None, :],
            g[None, :], b[None, :])


def _tc_heads(h, oW1, ob1, oW2p, ob2p, gW1, gb1, gW2p, gb2p):
    def body(h_ref, ow1_ref, ob1_ref, ow2_ref, ob2_ref,
             gw1_ref, gb1_ref, gw2_ref, gb2_ref, no_ref, go_ref):
        hv = h_ref[...]
        t = jnp.maximum(jnp.dot(hv, ow1_ref[...],
                                preferred_element_type=jnp.float32)
                        + ob1_ref[...], 0.0)
        no_ref[...] = jnp.dot(t, ow2_ref[...],
                              preferred_element_type=jnp.float32) + ob2_ref[...]
        gm = jnp.sum(hv, axis=0, keepdims=True) * (1.0 / N)
        tg = jnp.maximum(jnp.dot(gm, gw1_ref[...],
                                 preferred_element_type=jnp.float32)
                         + gb1_ref[...], 0.0)
        go_ref[...] = jnp.dot(tg, gw2_ref[...],
                              preferred_element_type=jnp.float32) + gb2_ref[...]
    return pl.pallas_call(
        body,
        out_shape=[jax.ShapeDtypeStruct((N, H), jnp.float32),
                   jax.ShapeDtypeStruct((1, H), jnp.float32)])(
            h, oW1, ob1[None, :], oW2p, ob2p[None, :],
            gW1, gb1[None, :], gW2p, gb2p[None, :])


def kernel(node_tensor, edge_tensor, Wn, bn, We, be, msg_W1, msg_b1, msg_W2,
           msg_b2, upd_W1, upd_b1, upd_W2, upd_b2, ln_g, ln_b, out_W1, out_b1,
           out_W2, out_b2, gr_W1, gr_b1, gr_W2, gr_b2, edge_index):
    E = edge_tensor.shape[0]
    e_pad = ((E + NW * CH - 1) // (NW * CH)) * (NW * CH)

    src = edge_index[0].astype(jnp.int32)
    dst = edge_index[1].astype(jnp.int32)
    src2 = jnp.pad(src, (0, e_pad - E)).reshape(NW, -1, CH)
    dst2 = jnp.pad(dst, (0, e_pad - E)).reshape(NW, -1, CH)
    dst2s = jnp.pad(dst, (0, e_pad - E), constant_values=N).reshape(NW, -1, CH)

    nt8 = jnp.pad(node_tensor, ((0, 0), (0, 8 - NI)))
    Wn8 = jnp.pad(Wn, ((0, 8 - NI), (0, 0)))
    et8 = jnp.pad(edge_tensor, ((0, e_pad - E), (0, 8 - EI)))
    We8 = jnp.pad(We, ((0, 8 - EI), (0, 0)))

    zero_rows = jnp.zeros((CH, H), jnp.float32)
    ones_rows = jnp.ones((CH, H), jnp.float32)

    oW2p = jnp.pad(out_W2, ((0, 0), (0, H - OUT)))
    ob2p = jnp.pad(out_b2, (0, H - OUT))
    gW2p = jnp.pad(gr_W2, ((0, 0), (0, H - OUT)))
    gb2p = jnp.pad(gr_b2, (0, H - OUT))

    h = _tc_encode_nodes(nt8, Wn8, bn)
    ef1 = _tc_encode_edges(et8, We8, be, e_pad)
    (cnt_parts,) = _sc_counts(dst2s, zero_rows, ones_rows)
    cnt_parts = cnt_parts[:, :, :8]

    for l in range(L):
        W1a = msg_W1[l, :H, :]
        W1b = msg_W1[l, H:2 * H, :]
        W1c = msg_W1[l, 2 * H:, :]
        hs_t, hd_t = _tc_proj(h, W1a, W1b)
        gs, gd = _sc_gather(hs_t, hd_t, src2, dst2, e_pad)
        m = _tc_edge_mlp(gs, gd, ef1, W1c, msg_W2[l], msg_b1[l], msg_b2[l],
                         e_pad)
        (parts,) = _sc_scatter(m, dst2s, zero_rows, e_pad)
        h = _tc_update(h, parts, cnt_parts, upd_W1[l, :H, :],
                       upd_W1[l, H:, :], upd_b1[l], upd_W2[l], upd_b2[l],
                       ln_g[l], ln_b[l])

    no_pad, go_pad = _tc_heads(h, out_W1, out_b1, oW2p, ob2p,
                               gr_W1, gr_b1, gW2p, gb2p)
    return no_pad[:, :OUT], go_pad[0, :OUT]


# speedup vs baseline: 3.5186x; 1.6622x over previous
import functools

import jax
import jax.numpy as jnp
from jax import lax
from jax.experimental import pallas as pl
from jax.experimental.pallas import tpu as pltpu
from jax.experimental.pallas import tpu_sc as plsc

N = 10000
H = 128
NI = 7
EI = 4
L = 4
OUT = 8

NC = 2
NS = 16
NW = NC * NS
CH = 128
NP = 10240
RPT = NP // NS


def _sc_gather(tbls, idxs, e_pad):
    K = e_pad // (NS * CH)
    RT = NP // NS
    mesh = plsc.VectorSubcoreMesh(
        core_axis_name="c", subcore_axis_name="s", num_cores=NC, num_subcores=NS)

    @functools.partial(
        pl.kernel, mesh=mesh,
        out_type=[jax.ShapeDtypeStruct((NC, e_pad, H), jnp.float32)],
        scratch_types=[pltpu.VMEM((K, CH), jnp.int32),
                       pltpu.VMEM((2, CH, H), jnp.float32),
                       pltpu.VMEM_SHARED((NP, H), jnp.float32),
                       [pltpu.SemaphoreType.DMA] * 2,
                       [pltpu.SemaphoreType.DMA] * 2])
    def k(tbls_hbm, idxs_hbm, gout_hbm, idx_v, rows, tbl_sp, gsem, wsem):
        cid = lax.axis_index("c")
        sid = lax.axis_index("s")
        base = sid * (K * CH)
        tsl = pl.ds(sid * RT, RT)
        pltpu.sync_copy(idxs_hbm.at[cid, sid], idx_v)
        pltpu.sync_copy(tbls_hbm.at[cid, tsl], tbl_sp.at[tsl])
        plsc.subcore_barrier()

        def start_g(c, b):
            pltpu.async_copy(tbl_sp.at[idx_v.at[c]], rows.at[b], gsem[b])

        def wait_g(b):
            pltpu.make_async_copy(tbl_sp.at[idx_v.at[0]], rows.at[b],
                                  gsem[b]).wait()

        def start_w(c, b):
            pltpu.async_copy(rows.at[b],
                             gout_hbm.at[cid, pl.ds(base + c * CH, CH)],
                             wsem[b])

        def wait_w(b):
            pltpu.make_async_copy(rows.at[b],
                                  gout_hbm.at[cid, pl.ds(base, CH)],
                                  wsem[b]).wait()

        start_g(0, 0)

        def body(gg, carry):
            for b in (0, 1):
                c = gg * 2 + b
                nb = 1 - b

                @pl.when(c >= 1)
                def _():
                    wait_w(nb)

                @pl.when(c + 1 < K)
                def _():
                    start_g(c + 1, nb)

                wait_g(b)
                start_w(c, b)
            return carry

        lax.fori_loop(0, K // 2, body, 0)
        wait_w(1)

    return k(tbls, idxs)


def _sc_scatter(m, dst2s, zero_rows, e_pad):
    K = e_pad // (NW * CH)
    mesh = plsc.VectorSubcoreMesh(
        core_axis_name="c", subcore_axis_name="s", num_cores=NC, num_subcores=NS)

    @functools.partial(
        pl.kernel, mesh=mesh,
        out_type=[jax.ShapeDtypeStruct((NC, NP, H), jnp.float32)],
        scratch_types=[pltpu.VMEM((K, CH), jnp.int32),
                       pltpu.VMEM((2, CH, H), jnp.float32),
                       pltpu.VMEM_SHARED((NP, H), jnp.float32),
                       [pltpu.SemaphoreType.DMA] * 2,
                       [pltpu.SemaphoreType.DMA] * 2])
    def k(m_hbm, dst_hbm, zero_hbm, agg_hbm, idx_d, mbuf, acc_sp, rsem, asem):
        cid = lax.axis_index("c")
        sid = lax.axis_index("s")
        wid = sid * NC + cid
        base = wid * (K * CH)
        tile0 = sid * RPT
        pltpu.sync_copy(dst_hbm.at[wid], idx_d)
        for j in range(RPT // CH):
            pltpu.sync_copy(zero_hbm, acc_sp.at[pl.ds(tile0 + j * CH, CH)])
        plsc.subcore_barrier()

        def start_r(c, b):
            pltpu.async_copy(m_hbm.at[pl.ds(base + c * CH, CH)], mbuf.at[b],
                             rsem[b])

        def wait_r(b):
            pltpu.make_async_copy(m_hbm.at[pl.ds(base, CH)], mbuf.at[b],
                                  rsem[b]).wait()

        def start_a(c, b):
            pltpu.async_copy(mbuf.at[b], acc_sp.at[idx_d.at[c]], asem[b],
                             add=True)

        def wait_a(b):
            pltpu.make_async_copy(mbuf.at[b], acc_sp.at[idx_d.at[0]],
                                  asem[b]).wait()

        start_r(0, 0)

        def body(gg, carry):
            for b in (0, 1):
                c = gg * 2 + b
                nb = 1 - b

                @pl.when(c >= 1)
                def _():
                    wait_a(nb)

                @pl.when(c + 1 < K)
                def _():
                    start_r(c + 1, nb)

                wait_r(b)
                start_a(c, b)
            return carry

        lax.fori_loop(0, K // 2, body, 0)
        wait_a(1)
        plsc.subcore_barrier()
        for j in range(RPT // CH):
            sl = pl.ds(tile0 + j * CH, CH)
            pltpu.sync_copy(acc_sp.at[sl], agg_hbm.at[cid, sl])

    return k(m, dst2s, zero_rows)


def _sc_counts(dst2s, zero_rows, ones_rows):
    K = dst2s.shape[1]
    mesh = plsc.VectorSubcoreMesh(
        core_axis_name="c", subcore_axis_name="s", num_cores=NC, num_subcores=NS)

    @functools.partial(
        pl.kernel, mesh=mesh,
        out_type=[jax.ShapeDtypeStruct((NC, NP, H), jnp.float32)],
        scratch_types=[pltpu.VMEM((K, CH), jnp.int32),
                       pltpu.VMEM((CH, H), jnp.float32),
                       pltpu.VMEM_SHARED((NP, H), jnp.float32),
                       pltpu.SemaphoreType.DMA])
    def k(dst_hbm, zero_hbm, ones_hbm, cnt_hbm, idx_d, obuf, acc_sp, asem):
        cid = lax.axis_index("c")
        sid = lax.axis_index("s")
        wid = sid * NC + cid
        tile0 = sid * RPT
        pltpu.sync_copy(dst_hbm.at[wid], idx_d)
        pltpu.sync_copy(ones_hbm, obuf)
        for j in range(RPT // CH):
            pltpu.sync_copy(zero_hbm, acc_sp.at[pl.ds(tile0 + j * CH, CH)])
        plsc.subcore_barrier()

        def chunk(c, carry):
            pltpu.async_copy(obuf, acc_sp.at[idx_d.at[c]], asem, add=True)
            return carry

        lax.fori_loop(0, K, chunk, 0)

        def drain(c, carry):
            pltpu.make_async_copy(obuf, acc_sp.at[idx_d.at[0]],
                                  asem).wait()
            return carry

        lax.fori_loop(0, K, drain, 0)
        plsc.subcore_barrier()
        for j in range(RPT // CH):
            sl = pl.ds(tile0 + j * CH, CH)
            pltpu.sync_copy(acc_sp.at[sl], cnt_hbm.at[cid, sl])

    return k(dst2s, zero_rows, ones_rows)


def _tc_encode_nodes(nt8, Wn8, bn):
    def body(x_ref, w_ref, b_ref, o_ref):
        o_ref[...] = jnp.dot(x_ref[...], w_ref[...],
                             preferred_element_type=jnp.float32) + b_ref[...]
    return pl.pallas_call(
        body, out_shape=jax.ShapeDtypeStruct((N, H), jnp.float32))(nt8, Wn8, bn[None, :])


def _tc_encode_edges(et8, We8, be, e_pad):
    BE = 2048
    def body(x_ref, w_ref, b_ref, o_ref):
        o_ref[...] = jnp.dot(x_ref[...], w_ref[...],
                             preferred_element_type=jnp.float32) + b_ref[...]
    return pl.pallas_call(
        body,
        grid=(e_pad // BE,),
        in_specs=[pl.BlockSpec((BE, 8), lambda i: (i, 0)),
                  pl.BlockSpec((8, H), lambda i: (0, 0)),
                  pl.BlockSpec((1, H), lambda i: (0, 0))],
        out_specs=pl.BlockSpec((BE, H), lambda i: (i, 0)),
        out_shape=jax.ShapeDtypeStruct((e_pad, H), jnp.float32))(et8, We8, be[None, :])


def _tc_proj(h, Wa, Wb):
    def body(h_ref, wa_ref, wb_ref, o_ref):
        hv = h_ref[...]
        o_ref[0, :N, :] = jnp.dot(hv, wa_ref[...],
                                  preferred_element_type=jnp.float32)
        o_ref[1, :N, :] = jnp.dot(hv, wb_ref[...],
                                  preferred_element_type=jnp.float32)
    return pl.pallas_call(
        body,
        out_shape=jax.ShapeDtypeStruct((2, NP, H), jnp.float32))(h, Wa, Wb)


def _tc_edge_mlp(gout, ef1, W1c, W2, b1, b2, e_pad):
    BE = 2048
    def body(gs_ref, gd_ref, ef_ref, w1_ref, w2_ref, b1_ref, b2_ref, m_ref):
        pre = (gs_ref[0] + gd_ref[0]
               + jnp.dot(ef_ref[...], w1_ref[...],
                         preferred_element_type=jnp.float32) + b1_ref[...])
        m_ref[...] = jnp.dot(jnp.maximum(pre, 0.0), w2_ref[...],
                             preferred_element_type=jnp.float32) + b2_ref[...]
    return pl.pallas_call(
        body,
        grid=(e_pad // BE,),
        in_specs=[pl.BlockSpec((1, BE, H), lambda i: (0, i, 0)),
                  pl.BlockSpec((1, BE, H), lambda i: (1, i, 0)),
                  pl.BlockSpec((BE, H), lambda i: (i, 0)),
                  pl.BlockSpec((H, H), lambda i: (0, 0)),
                  pl.BlockSpec((H, H), lambda i: (0, 0)),
                  pl.BlockSpec((1, H), lambda i: (0, 0)),
                  pl.BlockSpec((1, H), lambda i: (0, 0))],
        out_specs=pl.BlockSpec((BE, H), lambda i: (i, 0)),
        out_shape=jax.ShapeDtypeStruct((e_pad, H), jnp.float32))(
            gout, gout, ef1, W1c, W2, b1[None, :], b2[None, :])


def _tc_update(h, parts, cnt_parts, W1h, W1a, b1, W2, b2, g, b):
    def body(h_ref, p_ref, c_ref, w1h_ref, w1a_ref, b1_ref, w2_ref, b2_ref,
             g_ref, bb_ref, o_ref):
        hv = h_ref[...]
        agg = p_ref[0, :N, :] + p_ref[1, :N, :]
        cnt = c_ref[0, :N, 0:1] + c_ref[1, :N, 0:1]
        agg = agg * (1.0 / jnp.maximum(cnt, 1.0))
        pre = (jnp.dot(hv, w1h_ref[...], preferred_element_type=jnp.float32)
               + jnp.dot(agg, w1a_ref[...], preferred_element_type=jnp.float32)
               + b1_ref[...])
        u = jnp.dot(jnp.maximum(pre, 0.0), w2_ref[...],
                    preferred_element_type=jnp.float32) + b2_ref[...]
        x = hv + u
        mu = jnp.mean(x, axis=-1, keepdims=True)
        var = jnp.mean((x - mu) ** 2, axis=-1, keepdims=True)
        o_ref[...] = (x - mu) * lax.rsqrt(var + 1e-5) * g_ref[...] + bb_ref[...]
    return pl.pallas_call(
        body,
        out_shape=jax.ShapeDtypeStruct((N, H), jnp.float32))(
            h, parts, cnt_parts, W1h, W1a, b1[None, :], W2, b2[None, :],
            g[None, :], b[None, :])


def _tc_heads(h, oW1, ob1, oW2p, ob2p, gW1, gb1, gW2p, gb2p):
    def body(h_ref, ow1_ref, ob1_ref, ow2_ref, ob2_ref,
             gw1_ref, gb1_ref, gw2_ref, gb2_ref, no_ref, go_ref):
        hv = h_ref[...]
        t = jnp.maximum(jnp.dot(hv, ow1_ref[...],
                                preferred_element_type=jnp.float32)
                        + ob1_ref[...], 0.0)
        no_ref[...] = jnp.dot(t, ow2_ref[...],
                              preferred_element_type=jnp.float32) + ob2_ref[...]
        gm = jnp.sum(hv, axis=0, keepdims=True) * (1.0 / N)
        tg = jnp.maximum(jnp.dot(gm, gw1_ref[...],
                                 preferred_element_type=jnp.float32)
                         + gb1_ref[...], 0.0)
        go_ref[...] = jnp.dot(tg, gw2_ref[...],
                              preferred_element_type=jnp.float32) + gb2_ref[...]
    return pl.pallas_call(
        body,
        out_shape=[jax.ShapeDtypeStruct((N, H), jnp.float32),
                   jax.ShapeDtypeStruct((1, H), jnp.float32)])(
            h, oW1, ob1[None, :], oW2p, ob2p[None, :],
            gW1, gb1[None, :], gW2p, gb2p[None, :])


def kernel(node_tensor, edge_tensor, Wn, bn, We, be, msg_W1, msg_b1, msg_W2,
           msg_b2, upd_W1, upd_b1, upd_W2, upd_b2, ln_g, ln_b, out_W1, out_b1,
           out_W2, out_b2, gr_W1, gr_b1, gr_W2, gr_b2, edge_index):
    E = edge_tensor.shape[0]
    e_pad = ((E + NW * CH - 1) // (NW * CH)) * (NW * CH)

    src = edge_index[0].astype(jnp.int32)
    dst = edge_index[1].astype(jnp.int32)
    srcp = jnp.pad(src, (0, e_pad - E))
    dstp = jnp.pad(dst, (0, e_pad - E))
    idxs = jnp.stack([srcp.reshape(NS, -1, CH), dstp.reshape(NS, -1, CH)])
    dst2s = jnp.pad(dst, (0, e_pad - E), constant_values=N).reshape(NW, -1, CH)

    nt8 = jnp.pad(node_tensor, ((0, 0), (0, 8 - NI)))
    Wn8 = jnp.pad(Wn, ((0, 8 - NI), (0, 0)))
    et8 = jnp.pad(edge_tensor, ((0, e_pad - E), (0, 8 - EI)))
    We8 = jnp.pad(We, ((0, 8 - EI), (0, 0)))

    zero_rows = jnp.zeros((CH, H), jnp.float32)
    ones_rows = jnp.ones((CH, H), jnp.float32)

    oW2p = jnp.pad(out_W2, ((0, 0), (0, H - OUT)))
    ob2p = jnp.pad(out_b2, (0, H - OUT))
    gW2p = jnp.pad(gr_W2, ((0, 0), (0, H - OUT)))
    gb2p = jnp.pad(gr_b2, (0, H - OUT))

    h = _tc_encode_nodes(nt8, Wn8, bn)
    ef1 = _tc_encode_edges(et8, We8, be, e_pad)
    (cnt_parts,) = _sc_counts(dst2s, zero_rows, ones_rows)
    cnt_parts = cnt_parts[:, :, :8]

    for l in range(L):
        W1a = msg_W1[l, :H, :]
        W1b = msg_W1[l, H:2 * H, :]
        W1c = msg_W1[l, 2 * H:, :]
        tbls = _tc_proj(h, W1a, W1b)
        (gout,) = _sc_gather(tbls, idxs, e_pad)
        m = _tc_edge_mlp(gout, ef1, W1c, msg_W2[l], msg_b1[l], msg_b2[l],
                         e_pad)
        (parts,) = _sc_scatter(m, dst2s, zero_rows, e_pad)
        h = _tc_update(h, parts, cnt_parts, upd_W1[l, :H, :],
                       upd_W1[l, H:, :], upd_b1[l], upd_W2[l], upd_b2[l],
                       ln_g[l], ln_b[l])

    no_pad, go_pad = _tc_heads(h, out_W1, out_b1, oW2p, ob2p,
                               gr_W1, gr_b1, gW2p, gb2p)
    return no_pad[:, :OUT], go_pad[0, :OUT]


# speedup vs baseline: 3.8254x; 1.0872x over previous
import functools

import jax
import jax.numpy as jnp
from jax import lax
from jax.experimental import pallas as pl
from jax.experimental.pallas import tpu as pltpu
from jax.experimental.pallas import tpu_sc as plsc

N = 10000
H = 128
NI = 7
EI = 4
L = 4
OUT = 8

NC = 2
NS = 16
NW = NC * NS
CH = 128
NP = 10240
RPT = NP // NS


def _sc_gather(tbls, idxs, e_pad):
    K = e_pad // (NS * CH)
    RT = NP // NS
    mesh = plsc.VectorSubcoreMesh(
        core_axis_name="c", subcore_axis_name="s", num_cores=NC, num_subcores=NS)

    @functools.partial(
        pl.kernel, mesh=mesh,
        out_type=[jax.ShapeDtypeStruct((NC, e_pad, H), jnp.float32)],
        scratch_types=[pltpu.VMEM((K, CH), jnp.int32),
                       pltpu.VMEM((2, CH, H), jnp.float32),
                       pltpu.VMEM_SHARED((NP, H), jnp.float32),
                       [pltpu.SemaphoreType.DMA] * 2,
                       [pltpu.SemaphoreType.DMA] * 2])
    def k(tbls_hbm, idxs_hbm, gout_hbm, idx_v, rows, tbl_sp, gsem, wsem):
        cid = lax.axis_index("c")
        sid = lax.axis_index("s")
        base = sid * (K * CH)
        tsl = pl.ds(sid * RT, RT)
        pltpu.sync_copy(idxs_hbm.at[cid, sid], idx_v)
        pltpu.sync_copy(tbls_hbm.at[cid, tsl], tbl_sp.at[tsl])
        plsc.subcore_barrier()

        def start_g(c, b):
            pltpu.async_copy(tbl_sp.at[idx_v.at[c]], rows.at[b], gsem[b])

        def wait_g(b):
            pltpu.make_async_copy(tbl_sp.at[idx_v.at[0]], rows.at[b],
                                  gsem[b]).wait()

        def start_w(c, b):
            pltpu.async_copy(rows.at[b],
                             gout_hbm.at[cid, pl.ds(base + c * CH, CH)],
                             wsem[b])

        def wait_w(b):
            pltpu.make_async_copy(rows.at[b],
                                  gout_hbm.at[cid, pl.ds(base, CH)],
                                  wsem[b]).wait()

        start_g(0, 0)

        def body(gg, carry):
            for b in (0, 1):
                c = gg * 2 + b
                nb = 1 - b

                @pl.when(c >= 1)
                def _():
                    wait_w(nb)

                @pl.when(c + 1 < K)
                def _():
                    start_g(c + 1, nb)

                wait_g(b)
                start_w(c, b)
            return carry

        lax.fori_loop(0, K // 2, body, 0)
        wait_w(1)

    return k(tbls, idxs)


def _sc_scatter(m, dst2s, zero_rows, e_pad):
    K = e_pad // (NW * CH)
    mesh = plsc.VectorSubcoreMesh(
        core_axis_name="c", subcore_axis_name="s", num_cores=NC, num_subcores=NS)

    @functools.partial(
        pl.kernel, mesh=mesh,
        out_type=[jax.ShapeDtypeStruct((NC, NP, H), jnp.float32)],
        scratch_types=[pltpu.VMEM((K, CH), jnp.int32),
                       pltpu.VMEM((2, CH, H), jnp.float32),
                       pltpu.VMEM_SHARED((NP, H), jnp.float32),
                       [pltpu.SemaphoreType.DMA] * 2,
                       [pltpu.SemaphoreType.DMA] * 2])
    def k(m_hbm, dst_hbm, zero_hbm, agg_hbm, idx_d, mbuf, acc_sp, rsem, asem):
        cid = lax.axis_index("c")
        sid = lax.axis_index("s")
        wid = sid * NC + cid
        base = wid * (K * CH)
        tile0 = sid * RPT
        pltpu.sync_copy(dst_hbm.at[wid], idx_d)
        for j in range(RPT // CH):
            pltpu.sync_copy(zero_hbm, acc_sp.at[pl.ds(tile0 + j * CH, CH)])
        plsc.subcore_barrier()

        def start_r(c, b):
            pltpu.async_copy(m_hbm.at[pl.ds(base + c * CH, CH)], mbuf.at[b],
                             rsem[b])

        def wait_r(b):
            pltpu.make_async_copy(m_hbm.at[pl.ds(base, CH)], mbuf.at[b],
                                  rsem[b]).wait()

        def start_a(c, b):
            pltpu.async_copy(mbuf.at[b], acc_sp.at[idx_d.at[c]], asem[b],
                             add=True)

        def wait_a(b):
            pltpu.make_async_copy(mbuf.at[b], acc_sp.at[idx_d.at[0]],
                                  asem[b]).wait()

        start_r(0, 0)

        def body(gg, carry):
            for b in (0, 1):
                c = gg * 2 + b
                nb = 1 - b

                @pl.when(c >= 1)
                def _():
                    wait_a(nb)

                @pl.when(c + 1 < K)
                def _():
                    start_r(c + 1, nb)

                wait_r(b)
                start_a(c, b)
            return carry

        lax.fori_loop(0, K // 2, body, 0)
        wait_a(1)
        plsc.subcore_barrier()
        for j in range(RPT // CH):
            sl = pl.ds(tile0 + j * CH, CH)
            pltpu.sync_copy(acc_sp.at[sl], agg_hbm.at[cid, sl])

    return k(m, dst2s, zero_rows)


def _sc_counts(dst2s, zero_rows, ones_rows):
    K = dst2s.shape[1]
    mesh = plsc.VectorSubcoreMesh(
        core_axis_name="c", subcore_axis_name="s", num_cores=NC, num_subcores=NS)

    @functools.partial(
        pl.kernel, mesh=mesh,
        out_type=[jax.ShapeDtypeStruct((NC, NP, H), jnp.float32)],
        scratch_types=[pltpu.VMEM((K, CH), jnp.int32),
                       pltpu.VMEM((CH, H), jnp.float32),
                       pltpu.VMEM_SHARED((NP, H), jnp.float32),
                       pltpu.SemaphoreType.DMA])
    def k(dst_hbm, zero_hbm, ones_hbm, cnt_hbm, idx_d, obuf, acc_sp, asem):
        cid = lax.axis_index("c")
        sid = lax.axis_index("s")
        wid = sid * NC + cid
        tile0 = sid * RPT
        pltpu.sync_copy(dst_hbm.at[wid], idx_d)
        pltpu.sync_copy(ones_hbm, obuf)
        for j in range(RPT // CH):
            pltpu.sync_copy(zero_hbm, acc_sp.at[pl.ds(tile0 + j * CH, CH)])
        plsc.subcore_barrier()

        def chunk(c, carry):
            pltpu.async_copy(obuf, acc_sp.at[idx_d.at[c]], asem, add=True)
            return carry

        lax.fori_loop(0, K, chunk, 0)

        def drain(c, carry):
            pltpu.make_async_copy(obuf, acc_sp.at[idx_d.at[0]],
                                  asem).wait()
            return carry

        lax.fori_loop(0, K, drain, 0)
        plsc.subcore_barrier()
        for j in range(RPT // CH):
            sl = pl.ds(tile0 + j * CH, CH)
            pltpu.sync_copy(acc_sp.at[sl], cnt_hbm.at[cid, sl])

    return k(dst2s, zero_rows, ones_rows)


def _tc_encode_nodes(nt8, Wn8, bn):
    def body(x_ref, w_ref, b_ref, o_ref):
        o_ref[...] = jnp.dot(x_ref[...], w_ref[...],
                             preferred_element_type=jnp.float32) + b_ref[...]
    return pl.pallas_call(
        body, out_shape=jax.ShapeDtypeStruct((N, H), jnp.float32))(nt8, Wn8, bn[None, :])


def _tc_proj(h, Wa, Wb, We8, be, W1c, b1):
    def body(h_ref, wa_ref, wb_ref, we_ref, be_ref, w1c_ref, b1_ref,
             o_ref, wec_ref, bec_ref):
        hv = h_ref[...]
        o_ref[0, :N, :] = jnp.dot(hv, wa_ref[...],
                                  preferred_element_type=jnp.float32)
        o_ref[1, :N, :] = jnp.dot(hv, wb_ref[...],
                                  preferred_element_type=jnp.float32)
        wec_ref[...] = jnp.dot(we_ref[...], w1c_ref[...],
                               preferred_element_type=jnp.float32)
        bec_ref[...] = jnp.dot(be_ref[...], w1c_ref[...],
                               preferred_element_type=jnp.float32) + b1_ref[...]
    return pl.pallas_call(
        body,
        out_shape=[jax.ShapeDtypeStruct((2, NP, H), jnp.float32),
                   jax.ShapeDtypeStruct((8, H), jnp.float32),
                   jax.ShapeDtypeStruct((1, H), jnp.float32)])(
            h, Wa, Wb, We8, be[None, :], W1c, b1[None, :])


def _tc_edge_mlp(gout, et8, Wec, bec, W2, b2, e_pad):
    BE = 2048
    def body(gs_ref, gd_ref, et_ref, wec_ref, bec_ref, w2_ref, b2_ref, m_ref):
        pre = (gs_ref[0] + gd_ref[0]
               + jnp.dot(et_ref[...], wec_ref[...],
                         preferred_element_type=jnp.float32) + bec_ref[...])
        m_ref[...] = jnp.dot(jnp.maximum(pre, 0.0), w2_ref[...],
                             preferred_element_type=jnp.float32) + b2_ref[...]
    return pl.pallas_call(
        body,
        grid=(e_pad // BE,),
        in_specs=[pl.BlockSpec((1, BE, H), lambda i: (0, i, 0)),
                  pl.BlockSpec((1, BE, H), lambda i: (1, i, 0)),
                  pl.BlockSpec((BE, 8), lambda i: (i, 0)),
                  pl.BlockSpec((8, H), lambda i: (0, 0)),
                  pl.BlockSpec((1, H), lambda i: (0, 0)),
                  pl.BlockSpec((H, H), lambda i: (0, 0)),
                  pl.BlockSpec((1, H), lambda i: (0, 0))],
        out_specs=pl.BlockSpec((BE, H), lambda i: (i, 0)),
        out_shape=jax.ShapeDtypeStruct((e_pad, H), jnp.float32))(
            gout, gout, et8, Wec, bec, W2, b2[None, :])


def _tc_update(h, parts, cnt_parts, W1h, W1a, b1, W2, b2, g, b):
    def body(h_ref, p_ref, c_ref, w1h_ref, w1a_ref, b1_ref, w2_ref, b2_ref,
             g_ref, bb_ref, o_ref):
        hv = h_ref[...]
        agg = p_ref[0, :N, :] + p_ref[1, :N, :]
        cnt = c_ref[0, :N, 0:1] + c_ref[1, :N, 0:1]
        agg = agg * (1.0 / jnp.maximum(cnt, 1.0))
        pre = (jnp.dot(hv, w1h_ref[...], preferred_element_type=jnp.float32)
               + jnp.dot(agg, w1a_ref[...], preferred_element_type=jnp.float32)
               + b1_ref[...])
        u = jnp.dot(jnp.maximum(pre, 0.0), w2_ref[...],
                    preferred_element_type=jnp.float32) + b2_ref[...]
        x = hv + u
        mu = jnp.mean(x, axis=-1, keepdims=True)
        var = jnp.mean((x - mu) ** 2, axis=-1, keepdims=True)
        o_ref[...] = (x - mu) * lax.rsqrt(var + 1e-5) * g_ref[...] + bb_ref[...]
    return pl.pallas_call(
        body,
        out_shape=jax.ShapeDtypeStruct((N, H), jnp.float32))(
            h, parts, cnt_parts, W1h, W1a, b1[None, :], W2, b2[None, :],
            g[None, :], b[None, :])


def _tc_heads(h, oW1, ob1, oW2p, ob2p, gW1, gb1, gW2p, gb2p):
    def body(h_ref, ow1_ref, ob1_ref, ow2_ref, ob2_ref,
             gw1_ref, gb1_ref, gw2_ref, gb2_ref, no_ref, go_ref):
        hv = h_ref[...]
        t = jnp.maximum(jnp.dot(hv, ow1_ref[...],
                                preferred_element_type=jnp.float32)
                        + ob1_ref[...], 0.0)
        no_ref[...] = jnp.dot(t, ow2_ref[...],
                              preferred_element_type=jnp.float32) + ob2_ref[...]
        gm = jnp.sum(hv, axis=0, keepdims=True) * (1.0 / N)
        tg = jnp.maximum(jnp.dot(gm, gw1_ref[...],
                                 preferred_element_type=jnp.float32)
                         + gb1_ref[...], 0.0)
        go_ref[...] = jnp.dot(tg, gw2_ref[...],
                              preferred_element_type=jnp.float32) + gb2_ref[...]
    return pl.pallas_call(
        body,
        out_shape=[jax.ShapeDtypeStruct((N, H), jnp.float32),
                   jax.ShapeDtypeStruct((1, H), jnp.float32)])(
            h, oW1, ob1[None, :], oW2p, ob2p[None, :],
            gW1, gb1[None, :], gW2p, gb2p[None, :])


def kernel(node_tensor, edge_tensor, Wn, bn, We, be, msg_W1, msg_b1, msg_W2,
           msg_b2, upd_W1, upd_b1, upd_W2, upd_b2, ln_g, ln_b, out_W1, out_b1,
           out_W2, out_b2, gr_W1, gr_b1, gr_W2, gr_b2, edge_index):
    E = edge_tensor.shape[0]
    e_pad = ((E + NW * CH - 1) // (NW * CH)) * (NW * CH)

    src = edge_index[0].astype(jnp.int32)
    dst = edge_index[1].astype(jnp.int32)
    srcp = jnp.pad(src, (0, e_pad - E))
    dstp = jnp.pad(dst, (0, e_pad - E))
    idxs = jnp.stack([srcp.reshape(NS, -1, CH), dstp.reshape(NS, -1, CH)])
    dst2s = jnp.pad(dst, (0, e_pad - E), constant_values=N).reshape(NW, -1, CH)

    nt8 = jnp.pad(node_tensor, ((0, 0), (0, 8 - NI)))
    Wn8 = jnp.pad(Wn, ((0, 8 - NI), (0, 0)))
    et8 = jnp.pad(edge_tensor, ((0, e_pad - E), (0, 8 - EI)))
    We8 = jnp.pad(We, ((0, 8 - EI), (0, 0)))

    zero_rows = jnp.zeros((CH, H), jnp.float32)
    ones_rows = jnp.ones((CH, H), jnp.float32)

    oW2p = jnp.pad(out_W2, ((0, 0), (0, H - OUT)))
    ob2p = jnp.pad(out_b2, (0, H - OUT))
    gW2p = jnp.pad(gr_W2, ((0, 0), (0, H - OUT)))
    gb2p = jnp.pad(gr_b2, (0, H - OUT))

    h = _tc_encode_nodes(nt8, Wn8, bn)
    (cnt_parts,) = _sc_counts(dst2s, zero_rows, ones_rows)
    cnt_parts = cnt_parts[:, :, :8]

    for l in range(L):
        W1a = msg_W1[l, :H, :]
        W1b = msg_W1[l, H:2 * H, :]
        W1c = msg_W1[l, 2 * H:, :]
        tbls, Wec, bec = _tc_proj(h, W1a, W1b, We8, be, W1c, msg_b1[l])
        (gout,) = _sc_gather(tbls, idxs, e_pad)
        m = _tc_edge_mlp(gout, et8, Wec, bec, msg_W2[l], msg_b2[l],
                         e_pad)
        (parts,) = _sc_scatter(m, dst2s, zero_rows, e_pad)
        h = _tc_update(h, parts, cnt_parts, upd_W1[l, :H, :],
                       upd_W1[l, H:, :], upd_b1[l], upd_W2[l], upd_b2[l],
                       ln_g[l], ln_b[l])

    no_pad, go_pad = _tc_heads(h, out_W1, out_b1, oW2p, ob2p,
                               gr_W1, gr_b1, gW2p, gb2p)
    return no_pad[:, :OUT], go_pad[0, :OUT]


# speedup vs baseline: 3.8276x; 1.0006x over previous
import functools

import jax
import jax.numpy as jnp
from jax import lax
from jax.experimental import pallas as pl
from jax.experimental.pallas import tpu as pltpu
from jax.experimental.pallas import tpu_sc as plsc

N = 10000
H = 128
NI = 7
EI = 4
L = 4
OUT = 8

NC = 2
NS = 16
NW = NC * NS
CH = 128
NP = 10240
RPT = NP // NS


def _sc_gather(tbls, idxs, e_pad):
    K = e_pad // (NS * CH)
    RT = NP // NS
    assert K % 2 == 0
    mesh = plsc.VectorSubcoreMesh(
        core_axis_name="c", subcore_axis_name="s", num_cores=NC, num_subcores=NS)

    @functools.partial(
        pl.kernel, mesh=mesh,
        out_type=[jax.ShapeDtypeStruct((NC, e_pad, H), jnp.float32)],
        scratch_types=[pltpu.VMEM((K, CH), jnp.int32),
                       pltpu.VMEM((2, CH, H), jnp.float32),
                       pltpu.VMEM_SHARED((NP, H), jnp.float32),
                       [pltpu.SemaphoreType.DMA] * 2,
                       [pltpu.SemaphoreType.DMA] * 2])
    def k(tbls_hbm, idxs_hbm, gout_hbm, idx_v, rows, tbl_sp, gsem, wsem):
        cid = lax.axis_index("c")
        sid = lax.axis_index("s")
        base = sid * (K * CH)
        tsl = pl.ds(sid * RT, RT)
        pltpu.sync_copy(idxs_hbm.at[cid, sid], idx_v)
        pltpu.sync_copy(tbls_hbm.at[cid, tsl], tbl_sp.at[tsl])
        plsc.subcore_barrier()

        def start_g(c, b):
            pltpu.async_copy(tbl_sp.at[idx_v.at[c]], rows.at[b], gsem[b])

        def wait_g(b):
            pltpu.make_async_copy(tbl_sp.at[idx_v.at[0]], rows.at[b],
                                  gsem[b]).wait()

        def start_w(c, b):
            pltpu.async_copy(rows.at[b],
                             gout_hbm.at[cid, pl.ds(base + c * CH, CH)],
                             wsem[b])

        def wait_w(b):
            pltpu.make_async_copy(rows.at[b],
                                  gout_hbm.at[cid, pl.ds(base, CH)],
                                  wsem[b]).wait()

        start_g(0, 0)

        def body(gg, carry):
            for b in (0, 1):
                c = gg * 2 + b
                nb = 1 - b

                @pl.when(c >= 1)
                def _():
                    wait_w(nb)

                @pl.when(c + 1 < K)
                def _():
                    start_g(c + 1, nb)

                wait_g(b)
                start_w(c, b)
            return carry

        lax.fori_loop(0, K // 2, body, 0)
        wait_w(1)

    return k(tbls, idxs)


def _sc_scatter(mA, mB, dst2s, zero_rows, e_pad):
    K = e_pad // (NW * CH)
    EH = e_pad // 2
    mesh = plsc.VectorSubcoreMesh(
        core_axis_name="c", subcore_axis_name="s", num_cores=NC, num_subcores=NS)

    @functools.partial(
        pl.kernel, mesh=mesh,
        out_type=[jax.ShapeDtypeStruct((NC, NP, H), jnp.float32)],
        scratch_types=[pltpu.VMEM((K, CH), jnp.int32),
                       pltpu.VMEM((2, CH, H), jnp.float32),
                       pltpu.VMEM_SHARED((NP, H), jnp.float32),
                       [pltpu.SemaphoreType.DMA] * 2,
                       [pltpu.SemaphoreType.DMA] * 2])
    def k(mA_hbm, mB_hbm, dst_hbm, zero_hbm, agg_hbm, idx_d, mbuf, acc_sp,
          rsem, asem):
        cid = lax.axis_index("c")
        sid = lax.axis_index("s")
        wid = sid * NC + cid
        base = wid * (K * CH)
        tile0 = sid * RPT
        pltpu.sync_copy(dst_hbm.at[wid], idx_d)
        for j in range(RPT // CH):
            pltpu.sync_copy(zero_hbm, acc_sp.at[pl.ds(tile0 + j * CH, CH)])
        plsc.subcore_barrier()

        def run(m_hbm, lbase):
            def start_r(c, b):
                pltpu.async_copy(m_hbm.at[pl.ds(lbase + c * CH, CH)],
                                 mbuf.at[b], rsem[b])

            def wait_r(b):
                pltpu.make_async_copy(m_hbm.at[pl.ds(lbase, CH)], mbuf.at[b],
                                      rsem[b]).wait()

            def start_a(c, b):
                pltpu.async_copy(mbuf.at[b], acc_sp.at[idx_d.at[c]], asem[b],
                                 add=True)

            def wait_a(b):
                pltpu.make_async_copy(mbuf.at[b], acc_sp.at[idx_d.at[0]],
                                      asem[b]).wait()

            start_r(0, 0)

            def body(gg, carry):
                for b in (0, 1):
                    c = gg * 2 + b
                    nb = 1 - b

                    @pl.when(c >= 1)
                    def _():
                        wait_a(nb)

                    @pl.when(c + 1 < K)
                    def _():
                        start_r(c + 1, nb)

                    wait_r(b)
                    start_a(c, b)
                return carry

            lax.fori_loop(0, K // 2, body, 0)
            wait_a(1)

        @pl.when(wid < NW // 2)
        def _():
            run(mA_hbm, base)

        @pl.when(wid >= NW // 2)
        def _():
            run(mB_hbm, base - EH)

        plsc.subcore_barrier()
        for j in range(RPT // CH):
            sl = pl.ds(tile0 + j * CH, CH)
            pltpu.sync_copy(acc_sp.at[sl], agg_hbm.at[cid, sl])

    return k(mA, mB, dst2s, zero_rows)


def _sc_counts(dst2s, zero_rows, ones_rows):
    K = dst2s.shape[1]
    mesh = plsc.VectorSubcoreMesh(
        core_axis_name="c", subcore_axis_name="s", num_cores=NC, num_subcores=NS)

    @functools.partial(
        pl.kernel, mesh=mesh,
        out_type=[jax.ShapeDtypeStruct((NC, NP, H), jnp.float32)],
        scratch_types=[pltpu.VMEM((K, CH), jnp.int32),
                       pltpu.VMEM((CH, H), jnp.float32),
                       pltpu.VMEM_SHARED((NP, H), jnp.float32),
                       pltpu.SemaphoreType.DMA])
    def k(dst_hbm, zero_hbm, ones_hbm, cnt_hbm, idx_d, obuf, acc_sp, asem):
        cid = lax.axis_index("c")
        sid = lax.axis_index("s")
        wid = sid * NC + cid
        tile0 = sid * RPT
        pltpu.sync_copy(dst_hbm.at[wid], idx_d)
        pltpu.sync_copy(ones_hbm, obuf)
        for j in range(RPT // CH):
            pltpu.sync_copy(zero_hbm, acc_sp.at[pl.ds(tile0 + j * CH, CH)])
        plsc.subcore_barrier()

        def chunk(c, carry):
            pltpu.async_copy(obuf, acc_sp.at[idx_d.at[c]], asem, add=True)
            return carry

        lax.fori_loop(0, K, chunk, 0)

        def drain(c, carry):
            pltpu.make_async_copy(obuf, acc_sp.at[idx_d.at[0]],
                                  asem).wait()
            return carry

        lax.fori_loop(0, K, drain, 0)
        plsc.subcore_barrier()
        for j in range(RPT // CH):
            sl = pl.ds(tile0 + j * CH, CH)
            pltpu.sync_copy(acc_sp.at[sl], cnt_hbm.at[cid, sl])

    return k(dst2s, zero_rows, ones_rows)


def _tc_encode_nodes(nt8, Wn8, bn):
    def body(x_ref, w_ref, b_ref, o_ref):
        o_ref[...] = jnp.dot(x_ref[...], w_ref[...],
                             preferred_element_type=jnp.float32) + b_ref[...]
    return pl.pallas_call(
        body, out_shape=jax.ShapeDtypeStruct((N, H), jnp.float32))(nt8, Wn8, bn[None, :])


def _tc_proj(h, Wa, Wb, We8, be, W1c, b1):
    def body(h_ref, wa_ref, wb_ref, we_ref, be_ref, w1c_ref, b1_ref,
             o_ref, wec_ref, bec_ref):
        hv = h_ref[...]
        o_ref[0, :N, :] = jnp.dot(hv, wa_ref[...],
                                  preferred_element_type=jnp.float32)
        o_ref[1, :N, :] = jnp.dot(hv, wb_ref[...],
                                  preferred_element_type=jnp.float32)
        wec_ref[...] = jnp.dot(we_ref[...], w1c_ref[...],
                               preferred_element_type=jnp.float32)
        bec_ref[...] = jnp.dot(be_ref[...], w1c_ref[...],
                               preferred_element_type=jnp.float32) + b1_ref[...]
    return pl.pallas_call(
        body,
        out_shape=[jax.ShapeDtypeStruct((2, NP, H), jnp.float32),
                   jax.ShapeDtypeStruct((8, H), jnp.float32),
                   jax.ShapeDtypeStruct((1, H), jnp.float32)])(
            h, Wa, Wb, We8, be[None, :], W1c, b1[None, :])


def _tc_edge_mlp(gout, et8, Wec, bec, W2, b2, e_pad):
    BE = 2048
    def body(gs_ref, gd_ref, et_ref, wec_ref, bec_ref, w2_ref, b2_ref, m_ref):
        pre = (gs_ref[0] + gd_ref[0]
               + jnp.dot(et_ref[...], wec_ref[...],
                         preferred_element_type=jnp.float32) + bec_ref[...])
        m_ref[...] = jnp.dot(jnp.maximum(pre, 0.0), w2_ref[...],
                             preferred_element_type=jnp.float32) + b2_ref[...]
    return pl.pallas_call(
        body,
        grid=(e_pad // BE,),
        in_specs=[pl.BlockSpec((1, BE, H), lambda i: (0, i, 0)),
                  pl.BlockSpec((1, BE, H), lambda i: (1, i, 0)),
                  pl.BlockSpec((BE, 8), lambda i: (i, 0)),
                  pl.BlockSpec((8, H), lambda i: (0, 0)),
                  pl.BlockSpec((1, H), lambda i: (0, 0)),
                  pl.BlockSpec((H, H), lambda i: (0, 0)),
                  pl.BlockSpec((1, H), lambda i: (0, 0))],
        out_specs=pl.BlockSpec((BE, H), lambda i: (i, 0)),
        out_shape=jax.ShapeDtypeStruct((e_pad, H), jnp.float32))(
            gout, gout, et8, Wec, bec, W2, b2[None, :])


def _tc_update(h, parts, cnt_parts, W1h, W1a, b1, W2, b2, g, b):
    def body(h_ref, p_ref, c_ref, w1h_ref, w1a_ref, b1_ref, w2_ref, b2_ref,
             g_ref, bb_ref, o_ref):
        hv = h_ref[...]
        agg = p_ref[0, :N, :] + p_ref[1, :N, :]
        cnt = c_ref[0, :N, 0:1] + c_ref[1, :N, 0:1]
        agg = agg * (1.0 / jnp.maximum(cnt, 1.0))
        pre = (jnp.dot(hv, w1h_ref[...], preferred_element_type=jnp.float32)
               + jnp.dot(agg, w1a_ref[...], preferred_element_type=jnp.float32)
               + b1_ref[...])
        u = jnp.dot(jnp.maximum(pre, 0.0), w2_ref[...],
                    preferred_element_type=jnp.float32) + b2_ref[...]
        x = hv + u
        mu = jnp.mean(x, axis=-1, keepdims=True)
        var = jnp.mean((x - mu) ** 2, axis=-1, keepdims=True)
        o_ref[...] = (x - mu) * lax.rsqrt(var + 1e-5) * g_ref[...] + bb_ref[...]
    return pl.pallas_call(
        body,
        out_shape=jax.ShapeDtypeStruct((N, H), jnp.float32))(
            h, parts, cnt_parts, W1h, W1a, b1[None, :], W2, b2[None, :],
            g[None, :], b[None, :])


def _tc_heads(h, oW1, ob1, oW2p, ob2p, gW1, gb1, gW2p, gb2p):
    def body(h_ref, ow1_ref, ob1_ref, ow2_ref, ob2_ref,
             gw1_ref, gb1_ref, gw2_ref, gb2_ref, no_ref, go_ref):
        hv = h_ref[...]
        t = jnp.maximum(jnp.dot(hv, ow1_ref[...],
                                preferred_element_type=jnp.float32)
                        + ob1_ref[...], 0.0)
        no_ref[...] = jnp.dot(t, ow2_ref[...],
                              preferred_element_type=jnp.float32) + ob2_ref[...]
        gm = jnp.sum(hv, axis=0, keepdims=True) * (1.0 / N)
        tg = jnp.maximum(jnp.dot(gm, gw1_ref[...],
                                 preferred_element_type=jnp.float32)
                         + gb1_ref[...], 0.0)
        go_ref[...] = jnp.dot(tg, gw2_ref[...],
                              preferred_element_type=jnp.float32) + gb2_ref[...]
    return pl.pallas_call(
        body,
        out_shape=[jax.ShapeDtypeStruct((N, H), jnp.float32),
                   jax.ShapeDtypeStruct((1, H), jnp.float32)])(
            h, oW1, ob1[None, :], oW2p, ob2p[None, :],
            gW1, gb1[None, :], gW2p, gb2p[None, :])


def kernel(node_tensor, edge_tensor, Wn, bn, We, be, msg_W1, msg_b1, msg_W2,
           msg_b2, upd_W1, upd_b1, upd_W2, upd_b2, ln_g, ln_b, out_W1, out_b1,
           out_W2, out_b2, gr_W1, gr_b1, gr_W2, gr_b2, edge_index):
    E = edge_tensor.shape[0]
    e_pad = ((E + NW * CH - 1) // (NW * CH)) * (NW * CH)

    src = edge_index[0].astype(jnp.int32)
    dst = edge_index[1].astype(jnp.int32)
    srcp = jnp.pad(src, (0, e_pad - E))
    dstp = jnp.pad(dst, (0, e_pad - E))
    eh = e_pad // 2
    idxsA = jnp.stack([srcp[:eh].reshape(NS, -1, CH),
                       dstp[:eh].reshape(NS, -1, CH)])
    idxsB = jnp.stack([srcp[eh:].reshape(NS, -1, CH),
                       dstp[eh:].reshape(NS, -1, CH)])
    dst2s = jnp.pad(dst, (0, e_pad - E), constant_values=N).reshape(NW, -1, CH)

    nt8 = jnp.pad(node_tensor, ((0, 0), (0, 8 - NI)))
    Wn8 = jnp.pad(Wn, ((0, 8 - NI), (0, 0)))
    et8 = jnp.pad(edge_tensor, ((0, e_pad - E), (0, 8 - EI)))
    We8 = jnp.pad(We, ((0, 8 - EI), (0, 0)))

    zero_rows = jnp.zeros((CH, H), jnp.float32)
    ones_rows = jnp.ones((CH, H), jnp.float32)

    oW2p = jnp.pad(out_W2, ((0, 0), (0, H - OUT)))
    ob2p = jnp.pad(out_b2, (0, H - OUT))
    gW2p = jnp.pad(gr_W2, ((0, 0), (0, H - OUT)))
    gb2p = jnp.pad(gr_b2, (0, H - OUT))

    h = _tc_encode_nodes(nt8, Wn8, bn)
    (cnt_parts,) = _sc_counts(dst2s, zero_rows, ones_rows)
    cnt_parts = cnt_parts[:, :, :8]

    for l in range(L):
        W1a = msg_W1[l, :H, :]
        W1b = msg_W1[l, H:2 * H, :]
        W1c = msg_W1[l, 2 * H:, :]
        tbls, Wec, bec = _tc_proj(h, W1a, W1b, We8, be, W1c, msg_b1[l])
        (goutA,) = _sc_gather(tbls, idxsA, eh)
        (goutB,) = _sc_gather(tbls, idxsB, eh)
        mA = _tc_edge_mlp(goutA, et8[:eh], Wec, bec, msg_W2[l], msg_b2[l], eh)
        mB = _tc_edge_mlp(goutB, et8[eh:], Wec, bec, msg_W2[l], msg_b2[l], eh)
        (parts,) = _sc_scatter(mA, mB, dst2s, zero_rows, e_pad)
        h = _tc_update(h, parts, cnt_parts, upd_W1[l, :H, :],
                       upd_W1[l, H:, :], upd_b1[l], upd_W2[l], upd_b2[l],
                       ln_g[l], ln_b[l])

    no_pad, go_pad = _tc_heads(h, out_W1, out_b1, oW2p, ob2p,
                               gr_W1, gr_b1, gW2p, gb2p)
    return no_pad[:, :OUT], go_pad[0, :OUT]
